# Initial kernel scaffold; baseline (speedup 1.0000x reference)
import jax
import jax.numpy as jnp
from jax.experimental import pallas as pl


def kernel(x, edge_index, frontier_mask, aggregated_neighbors, W_neigh, b_neigh, W_root):
    raise NotImplementedError("write your pallas kernel here")


# speedup vs baseline: 13.0231x; 13.0231x over previous
import jax
import jax.numpy as jnp
from jax import lax
from jax.experimental import pallas as pl
from jax.experimental.pallas import tpu as pltpu
from jax.experimental.pallas import tpu_sc as plsc

_N = 10000
_E = 320000
_D = 128
_W = 144

_NC = 2
_NS = 16
_NWK = _NC * _NS
_CH = 128
_NCHK = _E // _CH
_CREM = _NCHK % _NWK
_CBASE = _NCHK // _NWK
_K = 2
_NP = 10240
_RPT = _NP // _NS
_BR = 2000


def _build_ext_body(xpad_ref, f_ref, xext_ref, flag_ref):
    f = f_ref[...]
    xz = xpad_ref[...] * (1.0 - f)
    col = lax.broadcasted_iota(jnp.int32, (_N, _W), 1)
    xext_ref[...] = jnp.where(col == _D, 1.0, jnp.where(col < _D, xz, 0.0))
    flag_ref[...] = jnp.max(f).reshape(1, 1)


def _sc_scatter_body(xext_hbm, src2d_hbm, dst2d_hbm, zeros_hbm, acc_out_hbm,
                     sidx, didx, rows, isems, gsems, ssems, acc):
    cid = lax.axis_index("c")
    sid = lax.axis_index("s")
    wid = sid * _NC + cid
    r0 = sid * _RPT
    pltpu.sync_copy(zeros_hbm.at[pl.ds(r0, _RPT)], acc.at[pl.ds(r0, _RPT)])
    plsc.subcore_barrier()

    c0 = wid * _CBASE + lax.min(wid, _CREM)
    cnt = _CBASE + jnp.where(wid < _CREM, 1, 0)

    def one_chunk(c, slot):
        i1 = pltpu.async_copy(src2d_hbm.at[c], sidx[slot], isems[slot])
        i2 = pltpu.async_copy(dst2d_hbm.at[c], didx[slot], isems[slot])
        i1.wait(); i2.wait()
        pltpu.async_copy(xext_hbm.at[sidx[slot]], rows[slot], gsems[slot]).wait()
        pltpu.async_copy(rows[slot], acc.at[didx[slot]], ssems[slot],
                         add=True).wait()

    def group(i, carry):
        c = c0 + i * _K
        iw, gd, sc = [], [], []
        for b in range(_K):
            iw.append((pltpu.async_copy(src2d_hbm.at[c + b], sidx[b], isems[b]),
                       pltpu.async_copy(dst2d_hbm.at[c + b], didx[b], isems[b])))
        for b in range(_K):
            iw[b][0].wait(); iw[b][1].wait()
            gd.append(pltpu.async_copy(xext_hbm.at[sidx[b]], rows[b], gsems[b]))
        for b in range(_K):
            gd[b].wait()
            sc.append(pltpu.async_copy(rows[b], acc.at[didx[b]], ssems[b],
                                       add=True))
        for b in range(_K):
            sc[b].wait()
        return carry

    ngrp = cnt // _K
    lax.fori_loop(0, ngrp, group, 0)

    def tail(i, carry):
        one_chunk(c0 + ngrp * _K + i, 0)
        return carry

    lax.fori_loop(0, cnt - ngrp * _K, tail, 0)

    plsc.subcore_barrier()
    pltpu.sync_copy(acc.at[pl.ds(r0, _RPT)], acc_out_hbm.at[cid, pl.ds(r0, _RPT)])


def _combine_body(acc_ref, x_ref, f_ref, agg_ref, wn_ref, b_ref, wr_ref,
                  flag_ref, out_ref):
    a = acc_ref[0] + acc_ref[1]
    summed = a[:, :_D]
    count = a[:, _D:_D + 1]
    mean = summed / jnp.maximum(count, 1.0)
    f = f_ref[...]
    xz = x_ref[...] * (1.0 - f)
    agg = agg_ref[...]
    use_hybrid = flag_ref[0, 0] > 0.0
    target = (jnp.sum(jnp.abs(agg), axis=1, keepdims=True) > 0.0) & use_hybrid
    neigh_in = jnp.where(target, agg, mean)
    root_in = jnp.where(target, 0.0, xz)
    out_ref[...] = (
        jnp.dot(neigh_in, wn_ref[...], preferred_element_type=jnp.float32)
        + b_ref[...]
        + jnp.dot(root_in, wr_ref[...], preferred_element_type=jnp.float32))


def kernel(x, edge_index, frontier_mask, aggregated_neighbors,
           W_neigh, b_neigh, W_root):
    f = frontier_mask.astype(jnp.float32).reshape(_N, 1)
    xpad = jnp.pad(x, ((0, 0), (0, _W - _D)))
    src2d = edge_index[0].reshape(_NCHK, _CH)
    dst2d = edge_index[1].reshape(_NCHK, _CH)
    zeros = jnp.zeros((_NP, _W), jnp.float32)
    b2 = b_neigh.reshape(1, _D)

    xext, flag = pl.pallas_call(
        _build_ext_body,
        out_shape=[jax.ShapeDtypeStruct((_N, _W), jnp.float32),
                   jax.ShapeDtypeStruct((1, 1), jnp.float32)],
    )(xpad, f)

    mesh = plsc.VectorSubcoreMesh(core_axis_name="c", subcore_axis_name="s")
    sc_scatter = pl.kernel(
        _sc_scatter_body,
        mesh=mesh,
        compiler_params=pltpu.CompilerParams(use_tc_tiling_on_sc=False),
        out_type=jax.ShapeDtypeStruct((_NC, _NP, _W), jnp.float32),
        scratch_types=[
            [pltpu.VMEM((_CH,), jnp.int32) for _ in range(_K)],
            [pltpu.VMEM((_CH,), jnp.int32) for _ in range(_K)],
            [pltpu.VMEM((_CH, _W), jnp.float32) for _ in range(_K)],
            [pltpu.SemaphoreType.DMA for _ in range(_K)],
            [pltpu.SemaphoreType.DMA for _ in range(_K)],
            [pltpu.SemaphoreType.DMA for _ in range(_K)],
            pltpu.VMEM_SHARED((_NP, _W), jnp.float32),
        ],
    )
    acc = sc_scatter(xext, src2d, dst2d, zeros)

    out = pl.pallas_call(
        _combine_body,
        grid=(_N // _BR,),
        in_specs=[
            pl.BlockSpec((_NC, _BR, _W), lambda i: (0, i, 0)),
            pl.BlockSpec((_BR, _D), lambda i: (i, 0)),
            pl.BlockSpec((_BR, 1), lambda i: (i, 0)),
            pl.BlockSpec((_BR, _D), lambda i: (i, 0)),
            pl.BlockSpec((_D, _D), lambda i: (0, 0)),
            pl.BlockSpec((1, _D), lambda i: (0, 0)),
            pl.BlockSpec((_D, _D), lambda i: (0, 0)),
            pl.BlockSpec((1, 1), lambda i: (0, 0)),
        ],
        out_specs=pl.BlockSpec((_BR, _D), lambda i: (i, 0)),
        out_shape=jax.ShapeDtypeStruct((_N, _D), jnp.float32),
    )(acc, x, f, aggregated_neighbors, W_neigh, b2, W_root, flag)
    return out
